# Initial kernel scaffold; baseline (speedup 1.0000x reference)
import jax
import jax.numpy as jnp
from jax.experimental import pallas as pl


def kernel(x, edge_index, edge_attr, batch, W1, b1, W2, b2, W3, b3, W4, b4):
    raise NotImplementedError("write your pallas kernel here")


# speedup vs baseline: 20.5064x; 20.5064x over previous
import functools

import jax
import jax.numpy as jnp
from jax import lax
from jax.experimental import pallas as pl
from jax.experimental.pallas import tpu as pltpu
from jax.experimental.pallas import tpu_sc as plsc

F32 = jnp.float32
I32 = jnp.int32

_SC_CP = pltpu.CompilerParams(needs_layout_passes=False,
                              use_tc_tiling_on_sc=False)

_NC = 2
_NS = 16
_L = 16


def _ceil_to(a, m):
    return (a + m - 1) // m * m


def _sc_deg(dst2d, attr2d, n_pad):
    rows = dst2d.shape[0]
    rows_per_core = rows // _NC
    rows_per_tile = rows_per_core // _NS
    brows = 8
    nblocks = rows_per_tile // brows
    npt = n_pad // _NS
    mesh = plsc.VectorSubcoreMesh(core_axis_name="c", subcore_axis_name="s")

    @functools.partial(
        pl.kernel, mesh=mesh, compiler_params=_SC_CP,
        out_type=jax.ShapeDtypeStruct((_NC, n_pad), F32),
        scratch_types=[
            pltpu.VMEM((brows, 128), I32),
            pltpu.VMEM((brows, 128), F32),
            pltpu.VMEM((npt,), F32),
            pltpu.VMEM_SHARED((n_pad,), F32),
            pltpu.SemaphoreType.DMA,
        ],
    )
    def k(dst_hbm, attr_hbm, out_hbm, idx_v, val_v, zero_v, acc_sh, sem):
        cid = lax.axis_index("c")
        sid = lax.axis_index("s")

        @pl.loop(0, npt, step=_L)
        def _(i):
            zero_v[pl.ds(i, _L)] = jnp.zeros((_L,), F32)

        pltpu.sync_copy(zero_v, acc_sh.at[pl.ds(sid * npt, npt)])
        plsc.subcore_barrier()

        base_row = cid * rows_per_core + sid * rows_per_tile

        @pl.loop(0, nblocks)
        def _(b):
            r0 = base_row + b * brows
            pltpu.sync_copy(dst_hbm.at[pl.ds(r0, brows)], idx_v)
            pltpu.sync_copy(attr_hbm.at[pl.ds(r0, brows)], val_v)
            for j in range(brows):
                pltpu.sync_copy(val_v.at[j], acc_sh.at[idx_v.at[j]], add=True)

        plsc.subcore_barrier()
        pltpu.sync_copy(acc_sh.at[pl.ds(sid * npt, npt)],
                        out_hbm.at[cid, pl.ds(sid * npt, npt)])

    return k(dst2d, attr2d)


def _sc_edge_scalar(src2d, dst2d, attr2d, u, n_pad):
    rows = src2d.shape[0]
    rows_per_core = rows // _NC
    rows_per_tile = rows_per_core // _NS
    brows = 8
    nblocks = rows_per_tile // brows
    npt = n_pad // _NS
    mesh = plsc.VectorSubcoreMesh(core_axis_name="c", subcore_axis_name="s")

    @functools.partial(
        pl.kernel, mesh=mesh, compiler_params=_SC_CP,
        out_type=jax.ShapeDtypeStruct((_NC, n_pad), F32),
        scratch_types=[
            pltpu.VMEM((brows, 128), I32),
            pltpu.VMEM((brows, 128), I32),
            pltpu.VMEM((brows, 128), F32),
            pltpu.VMEM((brows, 128), F32),
            pltpu.VMEM((n_pad,), F32),
            pltpu.VMEM((npt,), F32),
            pltpu.VMEM_SHARED((n_pad,), F32),
            pltpu.SemaphoreType.DMA,
        ],
    )
    def k(src_hbm, dst_hbm, attr_hbm, u_hbm, out_hbm,
          sidx_v, didx_v, av, mv, u_loc, zero_v, acc_sh, sem):
        cid = lax.axis_index("c")
        sid = lax.axis_index("s")

        pltpu.sync_copy(u_hbm, u_loc)

        @pl.loop(0, npt, step=_L)
        def _(i):
            zero_v[pl.ds(i, _L)] = jnp.zeros((_L,), F32)

        pltpu.sync_copy(zero_v, acc_sh.at[pl.ds(sid * npt, npt)])
        plsc.subcore_barrier()

        base_row = cid * rows_per_core + sid * rows_per_tile

        @pl.loop(0, nblocks)
        def _(b):
            r0 = base_row + b * brows
            pltpu.sync_copy(src_hbm.at[pl.ds(r0, brows)], sidx_v)
            pltpu.sync_copy(dst_hbm.at[pl.ds(r0, brows)], didx_v)
            pltpu.sync_copy(attr_hbm.at[pl.ds(r0, brows)], av)
            for j in range(brows):
                for c in range(0, 128, _L):
                    iv = sidx_v[j, pl.ds(c, _L)]
                    uv = plsc.load_gather(u_loc, [iv])
                    mv[j, pl.ds(c, _L)] = uv * av[j, pl.ds(c, _L)]
                pltpu.sync_copy(mv.at[j], acc_sh.at[didx_v.at[j]], add=True)

        plsc.subcore_barrier()
        pltpu.sync_copy(acc_sh.at[pl.ds(sid * npt, npt)],
                        out_hbm.at[cid, pl.ds(sid * npt, npt)])

    return k(src2d, dst2d, attr2d, u)


def _sc_edge_wide(src2d, dst2d, attr2d, y_flat, n_pad):
    rows = src2d.shape[0]
    rows_per_tile = rows // _NS
    brows = 4
    be = brows * 128
    nblocks = rows_per_tile // brows
    npt = n_pad // _NS
    zch = 112
    nz = npt // zch
    mesh = plsc.VectorSubcoreMesh(core_axis_name="c", subcore_axis_name="s")

    @functools.partial(
        pl.kernel, mesh=mesh, compiler_params=_SC_CP,
        out_type=jax.ShapeDtypeStruct((_NC, n_pad, 32), F32),
        scratch_types=[
            pltpu.VMEM((brows, 128), I32),
            pltpu.VMEM((brows, 128), I32),
            pltpu.VMEM((brows, 128), I32),
            pltpu.VMEM((be,), F32),
            pltpu.VMEM((be, 32), F32),
            pltpu.VMEM((zch, 32), F32),
            pltpu.VMEM_SHARED((n_pad, 32), F32),
            pltpu.SemaphoreType.DMA,
        ],
    )
    def k(src_hbm, dst_hbm, attr_hbm, y_hbm, out_hbm,
          sidx_v, gidx_v, didx_v, av, rows_v, zero_v, acc_sh, sem):
        cid = lax.axis_index("c")
        sid = lax.axis_index("s")

        @pl.loop(0, zch)
        def _(r):
            zero_v[r, pl.ds(0, _L)] = jnp.zeros((_L,), F32)
            zero_v[r, pl.ds(_L, _L)] = jnp.zeros((_L,), F32)

        @pl.loop(0, nz)
        def _(z):
            pltpu.sync_copy(zero_v, acc_sh.at[pl.ds(sid * npt + z * zch, zch)])

        plsc.subcore_barrier()

        base_row = sid * rows_per_tile

        @pl.loop(0, nblocks)
        def _(b):
            r0 = base_row + b * brows
            pltpu.sync_copy(src_hbm.at[pl.ds(r0, brows)], sidx_v)
            pltpu.sync_copy(dst_hbm.at[pl.ds(r0, brows)], didx_v)
            for j in range(brows):
                pltpu.sync_copy(attr_hbm.at[r0 + j], av.at[pl.ds(j * 128, 128)])
            for j in range(brows):
                for c in range(0, 128, _L):
                    gidx_v[j, pl.ds(c, _L)] = (
                        sidx_v[j, pl.ds(c, _L)] * 2 + cid)
            for j in range(brows):
                pltpu.async_copy(y_hbm.at[gidx_v.at[j]],
                                 rows_v.at[pl.ds(j * 128, 128)], sem).wait()

            @pl.loop(0, be, step=_L)
            def _(i0):
                a16 = av[pl.ds(i0, _L)]
                for l in range(_L):
                    a = jnp.full((_L,), a16[l], F32)
                    r = i0 + l
                    rows_v[r, pl.ds(0, _L)] = rows_v[r, pl.ds(0, _L)] * a
                    rows_v[r, pl.ds(_L, _L)] = rows_v[r, pl.ds(_L, _L)] * a

            for j in range(brows):
                pltpu.sync_copy(rows_v.at[pl.ds(j * 128, 128)],
                                acc_sh.at[didx_v.at[j]], add=True)

        plsc.subcore_barrier()
        pltpu.sync_copy(acc_sh.at[pl.ds(sid * npt, npt)],
                        out_hbm.at[cid, pl.ds(sid * npt, npt)])

    return k(src2d, dst2d, attr2d, y_flat)


_SQRT2_INV = 0.7071067811865476


def _gelu(v):
    return 0.5 * v * (1.0 + lax.erf(v * _SQRT2_INV))


def _tc_dinv_u(degp, x2d):
    def body(d_ref, x_ref, dinv_ref, u_ref):
        deg = 1.0 + d_ref[0] + d_ref[1]
        dinv = lax.rsqrt(deg)
        dinv_ref[...] = dinv
        u_ref[...] = dinv * x_ref[...]

    shp = x2d.shape
    return pl.pallas_call(
        body,
        out_shape=(jax.ShapeDtypeStruct(shp, F32), jax.ShapeDtypeStruct(shp, F32)),
    )(degp.reshape(2, *shp), x2d)


def _tc_s(tp, dinv2d, x2d):
    def body(t_ref, dinv_ref, x_ref, s_ref):
        dinv = dinv_ref[...]
        s_ref[...] = dinv * (t_ref[0] + t_ref[1] + dinv * x_ref[...])

    shp = x2d.shape
    return pl.pallas_call(
        body,
        out_shape=jax.ShapeDtypeStruct(shp, F32),
    )(tp.reshape(2, *shp), dinv2d, x2d)


def _tc_h1_y(s_col, dinv_col, w1, b1, w2, n_pad):
    blk = 512
    grid = n_pad // blk

    def body(s_ref, dinv_ref, w1_ref, b1_ref, w2_ref, y_ref):
        h1 = _gelu(s_ref[...] * w1_ref[...] + b1_ref[...])
        xw2 = jnp.dot(h1, w2_ref[...], preferred_element_type=F32,
                      precision=lax.Precision.HIGHEST)
        y_ref[...] = dinv_ref[...] * xw2

    return pl.pallas_call(
        body,
        grid=(grid,),
        in_specs=[
            pl.BlockSpec((blk, 1), lambda i: (i, 0)),
            pl.BlockSpec((blk, 1), lambda i: (i, 0)),
            pl.BlockSpec((1, 64), lambda i: (0, 0)),
            pl.BlockSpec((1, 64), lambda i: (0, 0)),
            pl.BlockSpec((64, 64), lambda i: (0, 0)),
        ],
        out_specs=pl.BlockSpec((blk, 64), lambda i: (i, 0)),
        out_shape=jax.ShapeDtypeStruct((n_pad, 64), F32),
    )(s_col, dinv_col, w1, b1, w2)


def _tc_head(acc0, acc1, y64, dinv_col, b2, batch_col, w3, b3, w4, b4, n_pad):
    blk = 512
    grid = n_pad // blk
    ng = 128

    def body(a0_ref, a1_ref, y_ref, dinv_ref, b2_ref, bat_ref,
             w3_ref, b3_ref, w4_ref, b4_ref, out_ref, pool_acc):
        i = pl.program_id(0)

        @pl.when(i == 0)
        def _():
            pool_acc[...] = jnp.zeros_like(pool_acc)

        acc = jnp.concatenate([a0_ref[...], a1_ref[...]], axis=1)
        conv2 = dinv_ref[...] * (acc + y_ref[...]) + b2_ref[...]
        h2 = _gelu(conv2)
        h2e = jnp.concatenate([h2, jnp.ones((blk, 1), F32)], axis=1)
        gids = lax.broadcasted_iota(I32, (blk, ng), 1)
        oh = (bat_ref[...] == gids).astype(F32)
        pool_acc[...] += lax.dot_general(
            oh, h2e, (((0,), (0,)), ((), ())),
            preferred_element_type=F32, precision=lax.Precision.HIGHEST)

        @pl.when(i == grid - 1)
        def _():
            cnt = jnp.maximum(pool_acc[:, 64:65], 1.0)
            pooled = pool_acc[:, :64] / cnt
            hp = _gelu(jnp.dot(pooled, w3_ref[...], preferred_element_type=F32,
                               precision=lax.Precision.HIGHEST) + b3_ref[...])
            out_ref[...] = jnp.dot(hp, w4_ref[...], preferred_element_type=F32,
                                   precision=lax.Precision.HIGHEST) + b4_ref[...]

    return pl.pallas_call(
        body,
        grid=(grid,),
        in_specs=[
            pl.BlockSpec((blk, 32), lambda i: (i, 0)),
            pl.BlockSpec((blk, 32), lambda i: (i, 0)),
            pl.BlockSpec((blk, 64), lambda i: (i, 0)),
            pl.BlockSpec((blk, 1), lambda i: (i, 0)),
            pl.BlockSpec((1, 64), lambda i: (0, 0)),
            pl.BlockSpec((blk, 1), lambda i: (i, 0)),
            pl.BlockSpec((64, 32), lambda i: (0, 0)),
            pl.BlockSpec((1, 32), lambda i: (0, 0)),
            pl.BlockSpec((32, 1), lambda i: (0, 0)),
            pl.BlockSpec((1, 1), lambda i: (0, 0)),
        ],
        out_specs=pl.BlockSpec((ng, 1), lambda i: (0, 0)),
        out_shape=jax.ShapeDtypeStruct((ng, 1), F32),
        scratch_shapes=[pltpu.VMEM((ng, 65), F32)],
    )(acc0, acc1, y64, dinv_col, b2, batch_col, w3, b3, w4, b4)


def kernel(x, edge_index, edge_attr, batch, W1, b1, W2, b2, W3, b3, W4, b4):
    n = x.shape[0]
    e = edge_index.shape[1]
    ng = 128

    n_pad = _ceil_to(n, 512)
    e_pad = _ceil_to(e, 32768)

    src = edge_index[0]
    dst = edge_index[1]
    pad_e = e_pad - e
    if pad_e:
        fill = (jnp.arange(pad_e, dtype=I32) * 997) % n
        src = jnp.concatenate([src, fill])
        dst = jnp.concatenate([dst, fill])
        edge_attr = jnp.concatenate([edge_attr, jnp.zeros((pad_e,), F32)])
    src2d = src.reshape(e_pad // 128, 128)
    dst2d = dst.reshape(e_pad // 128, 128)
    attr2d = edge_attr.reshape(e_pad // 128, 128)

    xf = jnp.pad(x[:, 0], (0, n_pad - n))
    x2d = xf.reshape(n_pad // 128, 128)
    batch_col = jnp.pad(batch, (0, n_pad - n), constant_values=ng).reshape(
        n_pad, 1)

    degp = _sc_deg(dst2d, attr2d, n_pad)
    dinv2d, u2d = _tc_dinv_u(degp, x2d)
    tp = _sc_edge_scalar(src2d, dst2d, attr2d, u2d.reshape(n_pad), n_pad)
    s2d = _tc_s(tp, dinv2d, x2d)

    s_col = s2d.reshape(n_pad, 1)
    dinv_col = dinv2d.reshape(n_pad, 1)
    y64 = _tc_h1_y(s_col, dinv_col, W1.reshape(1, 64), b1.reshape(1, 64), W2,
                   n_pad)
    y_flat = y64.reshape(2 * n_pad, 32)

    accs = _sc_edge_wide(src2d, dst2d, attr2d, y_flat, n_pad)

    out = _tc_head(accs[0], accs[1], y64, dinv_col, b2.reshape(1, 64),
                   batch_col, W3, b3.reshape(1, 32), W4, b4.reshape(1, 1),
                   n_pad)
    return out
